# baseline (speedup 1.0000x reference)
import functools

import jax
import jax.numpy as jnp
from jax import lax
from jax.experimental import pallas as pl
from jax.experimental.pallas import tpu as pltpu
from jax.experimental.pallas import tpu_sc as plsc

NC = 2
NS = 16
NW = NC * NS
LANES = 16
D = 128
B = 128
KBUF = 2


def _make_seg_sum(n_acc, n_chunks, with_deg):
    rows_per_tile = n_acc // NS
    row_chunks = rows_per_tile // B

    mesh = plsc.VectorSubcoreMesh(
        core_axis_name="c", subcore_axis_name="s",
        num_cores=NC, num_subcores=NS)

    out_type = [jax.ShapeDtypeStruct((NC, n_acc, D), jnp.float32)]
    if with_deg:
        out_type.append(jax.ShapeDtypeStruct((NC, n_acc), jnp.float32))

    K = KBUF
    scratch = (
        [pltpu.VMEM((B,), jnp.int32) for _ in range(K)]
        + [pltpu.VMEM((B,), jnp.int32) for _ in range(K)]
        + [pltpu.VMEM((B, D), jnp.float32) for _ in range(K)]
        + [pltpu.VMEM_SHARED((n_acc, D), jnp.float32)]
        + [pltpu.SemaphoreType.DMA for _ in range(3 * K)]
    )
    if with_deg:
        scratch += [
            pltpu.VMEM((B,), jnp.float32),
            pltpu.VMEM((rows_per_tile,), jnp.float32),
            pltpu.VMEM_SHARED((n_acc,), jnp.float32),
        ]

    def body(src_hbm, dst_hbm, y_hbm, acc_hbm, *rest):
        if with_deg:
            deg_hbm = rest[0]
            rest = rest[1:]
        srcb = rest[0:K]
        dstb = rest[K:2 * K]
        rows = rest[2 * K:3 * K]
        acc_sh = rest[3 * K]
        sems = rest[3 * K + 1:6 * K + 1]
        sems, semd, semg = sems[0:K], sems[K:2 * K], sems[2 * K:3 * K]
        if with_deg:
            ones_v, deg_v, deg_sh = rest[6 * K + 1:]
        rows0 = rows[0]
        cid = lax.axis_index("c")
        sid = lax.axis_index("s")
        wid = sid * NC + cid
        zeros16 = jnp.zeros((LANES,), jnp.float32)
        lanes_per_row = D // LANES

        base = wid * (n_chunks * B)

        def zb(i, carry):
            rows0[i // lanes_per_row,
                  pl.ds((i % lanes_per_row) * LANES, LANES)] = zeros16
            return carry
        lax.fori_loop(0, B * lanes_per_row, zb, 0)
        for j in range(row_chunks):
            pltpu.sync_copy(rows0, acc_sh.at[pl.ds(sid * rows_per_tile + j * B, B)])
        if with_deg:
            ones16 = jnp.ones((LANES,), jnp.float32)

            def zo(i, carry):
                ones_v[pl.ds(i * LANES, LANES)] = ones16
                return carry
            lax.fori_loop(0, B // LANES, zo, 0)

            def zd(i, carry):
                deg_v[pl.ds(i * LANES, LANES)] = zeros16
                return carry
            lax.fori_loop(0, rows_per_tile // LANES, zd, 0)
            pltpu.sync_copy(deg_v, deg_sh.at[pl.ds(sid * rows_per_tile, rows_per_tile)])
        plsc.subcore_barrier()

        def scatter(buf, dbuf):
            pltpu.sync_copy(buf, acc_sh.at[dbuf], add=True)
            if with_deg:
                pltpu.sync_copy(ones_v, deg_sh.at[dbuf], add=True)

        def step(it, carry):
            g0 = it * K
            for k in range(K):
                off = base + (g0 + k) * B
                pltpu.sync_copy(src_hbm.at[pl.ds(off, B)], srcb[k])
                pltpu.sync_copy(dst_hbm.at[pl.ds(off, B)], dstb[k])
                pltpu.async_copy(y_hbm.at[srcb[k]], rows[k], semg[k]).wait()
                scatter(rows[k], dstb[k])
            return carry
        lax.fori_loop(0, n_chunks // K, step, 0)

        plsc.subcore_barrier()

        r0 = sid * rows_per_tile
        pltpu.sync_copy(acc_sh.at[pl.ds(r0, rows_per_tile)],
                        acc_hbm.at[cid, pl.ds(r0, rows_per_tile)])
        if with_deg:
            pltpu.sync_copy(deg_sh.at[pl.ds(r0, rows_per_tile)],
                            deg_hbm.at[cid, pl.ds(r0, rows_per_tile)])

    return pl.kernel(body, out_type=out_type, mesh=mesh, scratch_types=scratch)


def _mm2(x, wa, wb, rows_blk=1000):
    n = x.shape[0]

    def body(x_ref, wa_ref, wb_ref, oa_ref, ob_ref):
        xb = x_ref[...]
        dn = (((1,), (1,)), ((), ()))
        oa_ref[...] = lax.dot_general(xb, wa_ref[...], dn,
                                      preferred_element_type=jnp.float32)
        ob_ref[...] = lax.dot_general(xb, wb_ref[...], dn,
                                      preferred_element_type=jnp.float32)

    return pl.pallas_call(
        body,
        grid=(n // rows_blk,),
        in_specs=[pl.BlockSpec((rows_blk, D), lambda i: (i, 0)),
                  pl.BlockSpec((D, D), lambda i: (0, 0)),
                  pl.BlockSpec((D, D), lambda i: (0, 0))],
        out_specs=[pl.BlockSpec((rows_blk, D), lambda i: (i, 0)),
                   pl.BlockSpec((rows_blk, D), lambda i: (i, 0))],
        out_shape=[jax.ShapeDtypeStruct((n, D), jnp.float32)] * 2,
    )(x, wa, wb)


def _deg_recip(deg_p):
    n_acc = deg_p.shape[1]

    def body(deg_ref, o_ref):
        deg = deg_ref[0] + deg_ref[1]
        o_ref[...] = (1.0 / jnp.maximum(deg, 1.0))[:, None]

    return pl.pallas_call(
        body,
        out_shape=jax.ShapeDtypeStruct((n_acc, 1), jnp.float32),
    )(deg_p)


def _combine_mm2(acc_p, recip, b, zr, wa, wb, rows_blk=1000):
    n = zr.shape[0]

    def body(acc_ref, recip_ref, b_ref, zr_ref, wa_ref, wb_ref, oa_ref, ob_ref):
        accsum = acc_ref[0] + acc_ref[1]
        h = accsum * recip_ref[...] + b_ref[...] + zr_ref[...]
        h = jnp.maximum(h, 0.0)
        dn = (((1,), (1,)), ((), ()))
        oa_ref[...] = lax.dot_general(h, wa_ref[...], dn,
                                      preferred_element_type=jnp.float32)
        ob_ref[...] = lax.dot_general(h, wb_ref[...], dn,
                                      preferred_element_type=jnp.float32)

    return pl.pallas_call(
        body,
        grid=(n // rows_blk,),
        in_specs=[pl.BlockSpec((NC, rows_blk, D), lambda i: (0, i, 0)),
                  pl.BlockSpec((rows_blk, 1), lambda i: (i, 0)),
                  pl.BlockSpec((1, D), lambda i: (0, 0)),
                  pl.BlockSpec((rows_blk, D), lambda i: (i, 0)),
                  pl.BlockSpec((D, D), lambda i: (0, 0)),
                  pl.BlockSpec((D, D), lambda i: (0, 0))],
        out_specs=[pl.BlockSpec((rows_blk, D), lambda i: (i, 0)),
                   pl.BlockSpec((rows_blk, D), lambda i: (i, 0))],
        out_shape=[jax.ShapeDtypeStruct((n, D), jnp.float32)] * 2,
    )(acc_p, recip, b, zr, wa, wb)


def _combine_final(acc_p, recip, b, zr, rows_blk=1000):
    n = zr.shape[0]

    def body(acc_ref, recip_ref, b_ref, zr_ref, o_ref):
        accsum = acc_ref[0] + acc_ref[1]
        o_ref[...] = accsum * recip_ref[...] + b_ref[...] + zr_ref[...]

    return pl.pallas_call(
        body,
        grid=(n // rows_blk,),
        in_specs=[pl.BlockSpec((NC, rows_blk, D), lambda i: (0, i, 0)),
                  pl.BlockSpec((rows_blk, 1), lambda i: (i, 0)),
                  pl.BlockSpec((1, D), lambda i: (0, 0)),
                  pl.BlockSpec((rows_blk, D), lambda i: (i, 0))],
        out_specs=pl.BlockSpec((rows_blk, D), lambda i: (i, 0)),
        out_shape=jax.ShapeDtypeStruct((n, D), jnp.float32),
    )(acc_p, recip, b, zr)


@jax.jit
def kernel(x, edge_index, W1_l, b1, W1_r, W2_l, b2, W2_r):
    n_nodes = x.shape[0]
    n_edges = edge_index.shape[1]

    per_w = -(-n_edges // (NW * B * KBUF)) * B * KBUF
    e_pad = per_w * NW
    n_chunks = per_w // B
    n_acc = -(-(n_nodes + 1) // (NS * B)) * NS * B

    src = edge_index[0].astype(jnp.int32)
    dst = edge_index[1].astype(jnp.int32)
    if e_pad > n_edges:
        pad = e_pad - n_edges
        src = jnp.concatenate([src, jnp.zeros((pad,), jnp.int32)])
        dst = jnp.concatenate([dst, jnp.full((pad,), n_nodes, jnp.int32)])


    seg_deg = _make_seg_sum(n_acc, n_chunks, with_deg=True)
    seg = _make_seg_sum(n_acc, n_chunks, with_deg=False)

    b1r = b1.reshape(1, D)
    b2r = b2.reshape(1, D)

    y1l, z1r = _mm2(x, W1_l, W1_r)
    acc1, deg_p = seg_deg(src, dst, y1l)
    recip = _deg_recip(deg_p)
    y2l, z2r = _combine_mm2(acc1, recip, b1r, z1r, W2_l, W2_r)
    (acc2,) = seg(src, dst, y2l)
    out = _combine_final(acc2, recip, b2r, z2r)
    return out


# speedup vs baseline: 1.1295x; 1.1295x over previous
import functools

import jax
import jax.numpy as jnp
from jax import lax
from jax.experimental import pallas as pl
from jax.experimental.pallas import tpu as pltpu
from jax.experimental.pallas import tpu_sc as plsc

NC = 2
NS = 16
NW = NC * NS
LANES = 16
D = 128
B = 256
KBUF = 1


def _make_seg_sum(n_acc, n_chunks, with_deg):
    rows_per_tile = n_acc // NS
    RB = 128
    row_chunks = rows_per_tile // RB

    mesh = plsc.VectorSubcoreMesh(
        core_axis_name="c", subcore_axis_name="s",
        num_cores=NC, num_subcores=NS)

    out_type = [jax.ShapeDtypeStruct((NC, n_acc, D), jnp.float32)]
    if with_deg:
        out_type.append(jax.ShapeDtypeStruct((NC, n_acc), jnp.float32))

    K = KBUF
    scratch = (
        [pltpu.VMEM((B,), jnp.int32) for _ in range(K)]
        + [pltpu.VMEM((B,), jnp.int32) for _ in range(K)]
        + [pltpu.VMEM((B, D), jnp.float32) for _ in range(K)]
        + [pltpu.VMEM_SHARED((n_acc, D), jnp.float32)]
        + [pltpu.SemaphoreType.DMA for _ in range(3 * K)]
    )
    if with_deg:
        scratch += [
            pltpu.VMEM((B,), jnp.float32),
            pltpu.VMEM((rows_per_tile,), jnp.float32),
            pltpu.VMEM_SHARED((n_acc,), jnp.float32),
        ]

    def body(src_hbm, dst_hbm, y_hbm, acc_hbm, *rest):
        if with_deg:
            deg_hbm = rest[0]
            rest = rest[1:]
        srcb = rest[0:K]
        dstb = rest[K:2 * K]
        rows = rest[2 * K:3 * K]
        acc_sh = rest[3 * K]
        sems = rest[3 * K + 1:6 * K + 1]
        sems, semd, semg = sems[0:K], sems[K:2 * K], sems[2 * K:3 * K]
        if with_deg:
            ones_v, deg_v, deg_sh = rest[6 * K + 1:]
        rows0 = rows[0]
        cid = lax.axis_index("c")
        sid = lax.axis_index("s")
        wid = sid * NC + cid
        zeros16 = jnp.zeros((LANES,), jnp.float32)
        lanes_per_row = D // LANES

        base = wid * (n_chunks * B)

        def zb(i, carry):
            rows0[i // lanes_per_row,
                  pl.ds((i % lanes_per_row) * LANES, LANES)] = zeros16
            return carry
        lax.fori_loop(0, RB * lanes_per_row, zb, 0)
        for j in range(row_chunks):
            pltpu.sync_copy(rows0.at[pl.ds(0, RB)],
                            acc_sh.at[pl.ds(sid * rows_per_tile + j * RB, RB)])
        if with_deg:
            ones16 = jnp.ones((LANES,), jnp.float32)

            def zo(i, carry):
                ones_v[pl.ds(i * LANES, LANES)] = ones16
                return carry
            lax.fori_loop(0, B // LANES, zo, 0)

            def zd(i, carry):
                deg_v[pl.ds(i * LANES, LANES)] = zeros16
                return carry
            lax.fori_loop(0, rows_per_tile // LANES, zd, 0)
            pltpu.sync_copy(deg_v, deg_sh.at[pl.ds(sid * rows_per_tile, rows_per_tile)])
        plsc.subcore_barrier()

        def scatter(buf, dbuf):
            pltpu.sync_copy(buf, acc_sh.at[dbuf], add=True)
            if with_deg:
                pltpu.sync_copy(ones_v, deg_sh.at[dbuf], add=True)

        def step(it, carry):
            g0 = it * K
            sd, dd, gd = [], [], []
            for k in range(K):
                off = base + (g0 + k) * B
                sd.append(pltpu.async_copy(
                    src_hbm.at[pl.ds(off, B)], srcb[k], sems[k]))
                dd.append(pltpu.async_copy(
                    dst_hbm.at[pl.ds(off, B)], dstb[k], semd[k]))
            for k in range(K):
                sd[k].wait()
                gd.append(pltpu.async_copy(
                    y_hbm.at[srcb[k]], rows[k], semg[k]))
            for k in range(K):
                gd[k].wait()
                dd[k].wait()
                scatter(rows[k], dstb[k])
            return carry
        lax.fori_loop(0, n_chunks // K, step, 0)

        plsc.subcore_barrier()

        for j in range(row_chunks):
            r0 = sid * rows_per_tile + j * RB
            pltpu.sync_copy(acc_sh.at[pl.ds(r0, RB)], rows0.at[pl.ds(0, RB)])
            pltpu.sync_copy(rows0.at[pl.ds(0, RB)], acc_hbm.at[cid, pl.ds(r0, RB)])
        if with_deg:
            r0 = sid * rows_per_tile
            pltpu.sync_copy(deg_sh.at[pl.ds(r0, rows_per_tile)], deg_v)
            pltpu.sync_copy(deg_v, deg_hbm.at[cid, pl.ds(r0, rows_per_tile)])

    return pl.kernel(body, out_type=out_type, mesh=mesh, scratch_types=scratch)


def _mm2(x, wa, wb, rows_blk=1000):
    n = x.shape[0]

    def body(x_ref, wa_ref, wb_ref, oa_ref, ob_ref):
        xb = x_ref[...]
        dn = (((1,), (1,)), ((), ()))
        oa_ref[...] = lax.dot_general(xb, wa_ref[...], dn,
                                      preferred_element_type=jnp.float32)
        ob_ref[...] = lax.dot_general(xb, wb_ref[...], dn,
                                      preferred_element_type=jnp.float32)

    return pl.pallas_call(
        body,
        grid=(n // rows_blk,),
        in_specs=[pl.BlockSpec((rows_blk, D), lambda i: (i, 0)),
                  pl.BlockSpec((D, D), lambda i: (0, 0)),
                  pl.BlockSpec((D, D), lambda i: (0, 0))],
        out_specs=[pl.BlockSpec((rows_blk, D), lambda i: (i, 0)),
                   pl.BlockSpec((rows_blk, D), lambda i: (i, 0))],
        out_shape=[jax.ShapeDtypeStruct((n, D), jnp.float32)] * 2,
    )(x, wa, wb)


def _deg_recip(deg_p):
    n_acc = deg_p.shape[1]

    def body(deg_ref, o_ref):
        deg = deg_ref[0] + deg_ref[1]
        o_ref[...] = (1.0 / jnp.maximum(deg, 1.0))[:, None]

    return pl.pallas_call(
        body,
        out_shape=jax.ShapeDtypeStruct((n_acc, 1), jnp.float32),
    )(deg_p)


def _combine_mm2(acc_p, recip, b, zr, wa, wb, rows_blk=1000):
    n = zr.shape[0]

    def body(acc_ref, recip_ref, b_ref, zr_ref, wa_ref, wb_ref, oa_ref, ob_ref):
        accsum = acc_ref[0] + acc_ref[1]
        h = accsum * recip_ref[...] + b_ref[...] + zr_ref[...]
        h = jnp.maximum(h, 0.0)
        dn = (((1,), (1,)), ((), ()))
        oa_ref[...] = lax.dot_general(h, wa_ref[...], dn,
                                      preferred_element_type=jnp.float32)
        ob_ref[...] = lax.dot_general(h, wb_ref[...], dn,
                                      preferred_element_type=jnp.float32)

    return pl.pallas_call(
        body,
        grid=(n // rows_blk,),
        in_specs=[pl.BlockSpec((NC, rows_blk, D), lambda i: (0, i, 0)),
                  pl.BlockSpec((rows_blk, 1), lambda i: (i, 0)),
                  pl.BlockSpec((1, D), lambda i: (0, 0)),
                  pl.BlockSpec((rows_blk, D), lambda i: (i, 0)),
                  pl.BlockSpec((D, D), lambda i: (0, 0)),
                  pl.BlockSpec((D, D), lambda i: (0, 0))],
        out_specs=[pl.BlockSpec((rows_blk, D), lambda i: (i, 0)),
                   pl.BlockSpec((rows_blk, D), lambda i: (i, 0))],
        out_shape=[jax.ShapeDtypeStruct((n, D), jnp.float32)] * 2,
    )(acc_p, recip, b, zr, wa, wb)


def _combine_final(acc_p, recip, b, zr, rows_blk=1000):
    n = zr.shape[0]

    def body(acc_ref, recip_ref, b_ref, zr_ref, o_ref):
        accsum = acc_ref[0] + acc_ref[1]
        o_ref[...] = accsum * recip_ref[...] + b_ref[...] + zr_ref[...]

    return pl.pallas_call(
        body,
        grid=(n // rows_blk,),
        in_specs=[pl.BlockSpec((NC, rows_blk, D), lambda i: (0, i, 0)),
                  pl.BlockSpec((rows_blk, 1), lambda i: (i, 0)),
                  pl.BlockSpec((1, D), lambda i: (0, 0)),
                  pl.BlockSpec((rows_blk, D), lambda i: (i, 0))],
        out_specs=pl.BlockSpec((rows_blk, D), lambda i: (i, 0)),
        out_shape=jax.ShapeDtypeStruct((n, D), jnp.float32),
    )(acc_p, recip, b, zr)


@jax.jit
def kernel(x, edge_index, W1_l, b1, W1_r, W2_l, b2, W2_r):
    n_nodes = x.shape[0]
    n_edges = edge_index.shape[1]

    per_w = -(-n_edges // (NW * B * KBUF)) * B * KBUF
    e_pad = per_w * NW
    n_chunks = per_w // B
    n_acc = -(-(n_nodes + 1) // (NS * 128)) * NS * 128

    src = edge_index[0].astype(jnp.int32)
    dst = edge_index[1].astype(jnp.int32)
    if e_pad > n_edges:
        pad = e_pad - n_edges
        src = jnp.concatenate([src, jnp.zeros((pad,), jnp.int32)])
        dst = jnp.concatenate([dst, jnp.full((pad,), n_nodes, jnp.int32)])


    seg_deg = _make_seg_sum(n_acc, n_chunks, with_deg=True)
    seg = _make_seg_sum(n_acc, n_chunks, with_deg=False)

    b1r = b1.reshape(1, D)
    b2r = b2.reshape(1, D)

    y1l, z1r = _mm2(x, W1_l, W1_r)
    acc1, deg_p = seg_deg(src, dst, y1l)
    recip = _deg_recip(deg_p)
    y2l, z2r = _combine_mm2(acc1, recip, b1r, z1r, W2_l, W2_r)
    (acc2,) = seg(src, dst, y2l)
    out = _combine_final(acc2, recip, b2r, z2r)
    return out


# speedup vs baseline: 1.6787x; 1.4862x over previous
import functools

import jax
import jax.numpy as jnp
from jax import lax
from jax.experimental import pallas as pl
from jax.experimental.pallas import tpu as pltpu
from jax.experimental.pallas import tpu_sc as plsc

NC = 2
NS = 16
NW = NC * NS
LANES = 16
D = 128
B = 64
KBUF = 1


def _make_seg_sum(n_acc, n_chunks, with_deg):
    rows_per_tile = n_acc // NS
    RB = 64
    row_chunks = rows_per_tile // RB

    mesh = plsc.VectorSubcoreMesh(
        core_axis_name="c", subcore_axis_name="s",
        num_cores=NC, num_subcores=NS)

    out_type = [jax.ShapeDtypeStruct((NC, n_acc, D), jnp.float32)]
    if with_deg:
        out_type.append(jax.ShapeDtypeStruct((NC, n_acc), jnp.float32))

    K = KBUF
    scratch = (
        [pltpu.VMEM((B,), jnp.int32) for _ in range(K)]
        + [pltpu.VMEM((B,), jnp.int32) for _ in range(K)]
        + [pltpu.VMEM((B, D), jnp.float32) for _ in range(K)]
        + [pltpu.VMEM_SHARED((n_acc, D), jnp.float32)]
        + [pltpu.SemaphoreType.DMA for _ in range(3 * K)]
    )
    if with_deg:
        scratch += [
            pltpu.VMEM((B,), jnp.float32),
            pltpu.VMEM((rows_per_tile,), jnp.float32),
            pltpu.VMEM_SHARED((n_acc,), jnp.float32),
        ]

    def body(src_hbm, dst_hbm, y_hbm, acc_hbm, *rest):
        if with_deg:
            deg_hbm = rest[0]
            rest = rest[1:]
        srcb = rest[0:K]
        dstb = rest[K:2 * K]
        rows = rest[2 * K:3 * K]
        acc_sh = rest[3 * K]
        sems = rest[3 * K + 1:6 * K + 1]
        sems, semd, semg = sems[0:K], sems[K:2 * K], sems[2 * K:3 * K]
        if with_deg:
            ones_v, deg_v, deg_sh = rest[6 * K + 1:]
        rows0 = rows[0]
        cid = lax.axis_index("c")
        sid = lax.axis_index("s")
        wid = sid * NC + cid
        zeros16 = jnp.zeros((LANES,), jnp.float32)
        lanes_per_row = D // LANES

        base = wid * (n_chunks * B)

        def zb(i, carry):
            rows0[i // lanes_per_row,
                  pl.ds((i % lanes_per_row) * LANES, LANES)] = zeros16
            return carry
        lax.fori_loop(0, RB * lanes_per_row, zb, 0)
        for j in range(row_chunks):
            pltpu.sync_copy(rows0, acc_sh.at[pl.ds(sid * rows_per_tile + j * RB, RB)])
        if with_deg:
            ones16 = jnp.ones((LANES,), jnp.float32)

            def zo(i, carry):
                ones_v[pl.ds(i * LANES, LANES)] = ones16
                return carry
            lax.fori_loop(0, B // LANES, zo, 0)

            def zd(i, carry):
                deg_v[pl.ds(i * LANES, LANES)] = zeros16
                return carry
            lax.fori_loop(0, rows_per_tile // LANES, zd, 0)
            pltpu.sync_copy(deg_v, deg_sh.at[pl.ds(sid * rows_per_tile, rows_per_tile)])
        plsc.subcore_barrier()

        def scatter(buf, dbuf):
            pltpu.sync_copy(buf, acc_sh.at[dbuf], add=True)
            if with_deg:
                pltpu.sync_copy(ones_v, deg_sh.at[dbuf], add=True)

        def step(it, carry):
            g0 = it * K
            sd, dd, gd = [], [], []
            for k in range(K):
                off = base + (g0 + k) * B
                sd.append(pltpu.async_copy(
                    src_hbm.at[pl.ds(off, B)], srcb[k], sems[k]))
                dd.append(pltpu.async_copy(
                    dst_hbm.at[pl.ds(off, B)], dstb[k], semd[k]))
            for k in range(K):
                sd[k].wait()
                gd.append(pltpu.async_copy(
                    y_hbm.at[srcb[k]], rows[k], semg[k]))
            for k in range(K):
                gd[k].wait()
                dd[k].wait()
                scatter(rows[k], dstb[k])
            return carry
        lax.fori_loop(0, n_chunks // K, step, 0)

        plsc.subcore_barrier()

        for j in range(row_chunks):
            r0 = sid * rows_per_tile + j * RB
            pltpu.sync_copy(acc_sh.at[pl.ds(r0, RB)], rows0)
            pltpu.sync_copy(rows0, acc_hbm.at[cid, pl.ds(r0, RB)])
        if with_deg:
            r0 = sid * rows_per_tile
            pltpu.sync_copy(deg_sh.at[pl.ds(r0, rows_per_tile)], deg_v)
            pltpu.sync_copy(deg_v, deg_hbm.at[cid, pl.ds(r0, rows_per_tile)])

    return pl.kernel(body, out_type=out_type, mesh=mesh, scratch_types=scratch)


def _mm2(x, wa, wb, rows_blk=1000):
    n = x.shape[0]

    def body(x_ref, wa_ref, wb_ref, oa_ref, ob_ref):
        xb = x_ref[...]
        dn = (((1,), (1,)), ((), ()))
        oa_ref[...] = lax.dot_general(xb, wa_ref[...], dn,
                                      preferred_element_type=jnp.float32)
        ob_ref[...] = lax.dot_general(xb, wb_ref[...], dn,
                                      preferred_element_type=jnp.float32)

    return pl.pallas_call(
        body,
        grid=(n // rows_blk,),
        in_specs=[pl.BlockSpec((rows_blk, D), lambda i: (i, 0)),
                  pl.BlockSpec((D, D), lambda i: (0, 0)),
                  pl.BlockSpec((D, D), lambda i: (0, 0))],
        out_specs=[pl.BlockSpec((rows_blk, D), lambda i: (i, 0)),
                   pl.BlockSpec((rows_blk, D), lambda i: (i, 0))],
        out_shape=[jax.ShapeDtypeStruct((n, D), jnp.float32)] * 2,
    )(x, wa, wb)


def _deg_recip(deg_p):
    n_acc = deg_p.shape[1]

    def body(deg_ref, o_ref):
        deg = deg_ref[0] + deg_ref[1]
        o_ref[...] = (1.0 / jnp.maximum(deg, 1.0))[:, None]

    return pl.pallas_call(
        body,
        out_shape=jax.ShapeDtypeStruct((n_acc, 1), jnp.float32),
    )(deg_p)


def _combine_mm2(acc_p, recip, b, zr, wa, wb, rows_blk=1000):
    n = zr.shape[0]

    def body(acc_ref, recip_ref, b_ref, zr_ref, wa_ref, wb_ref, oa_ref, ob_ref):
        accsum = acc_ref[0] + acc_ref[1]
        h = accsum * recip_ref[...] + b_ref[...] + zr_ref[...]
        h = jnp.maximum(h, 0.0)
        dn = (((1,), (1,)), ((), ()))
        oa_ref[...] = lax.dot_general(h, wa_ref[...], dn,
                                      preferred_element_type=jnp.float32)
        ob_ref[...] = lax.dot_general(h, wb_ref[...], dn,
                                      preferred_element_type=jnp.float32)

    return pl.pallas_call(
        body,
        grid=(n // rows_blk,),
        in_specs=[pl.BlockSpec((NC, rows_blk, D), lambda i: (0, i, 0)),
                  pl.BlockSpec((rows_blk, 1), lambda i: (i, 0)),
                  pl.BlockSpec((1, D), lambda i: (0, 0)),
                  pl.BlockSpec((rows_blk, D), lambda i: (i, 0)),
                  pl.BlockSpec((D, D), lambda i: (0, 0)),
                  pl.BlockSpec((D, D), lambda i: (0, 0))],
        out_specs=[pl.BlockSpec((rows_blk, D), lambda i: (i, 0)),
                   pl.BlockSpec((rows_blk, D), lambda i: (i, 0))],
        out_shape=[jax.ShapeDtypeStruct((n, D), jnp.float32)] * 2,
    )(acc_p, recip, b, zr, wa, wb)


def _combine_final(acc_p, recip, b, zr, rows_blk=1000):
    n = zr.shape[0]

    def body(acc_ref, recip_ref, b_ref, zr_ref, o_ref):
        accsum = acc_ref[0] + acc_ref[1]
        o_ref[...] = accsum * recip_ref[...] + b_ref[...] + zr_ref[...]

    return pl.pallas_call(
        body,
        grid=(n // rows_blk,),
        in_specs=[pl.BlockSpec((NC, rows_blk, D), lambda i: (0, i, 0)),
                  pl.BlockSpec((rows_blk, 1), lambda i: (i, 0)),
                  pl.BlockSpec((1, D), lambda i: (0, 0)),
                  pl.BlockSpec((rows_blk, D), lambda i: (i, 0))],
        out_specs=pl.BlockSpec((rows_blk, D), lambda i: (i, 0)),
        out_shape=jax.ShapeDtypeStruct((n, D), jnp.float32),
    )(acc_p, recip, b, zr)


@jax.jit
def kernel(x, edge_index, W1_l, b1, W1_r, W2_l, b2, W2_r):
    n_nodes = x.shape[0]
    n_edges = edge_index.shape[1]

    per_w = -(-n_edges // (NW * B * KBUF)) * B * KBUF
    e_pad = per_w * NW
    n_chunks = per_w // B
    n_acc = -(-(n_nodes + 1) // (NS * 128)) * NS * 128

    src = edge_index[0].astype(jnp.int32)
    dst = edge_index[1].astype(jnp.int32)
    if e_pad > n_edges:
        pad = e_pad - n_edges
        src = jnp.concatenate([src, jnp.zeros((pad,), jnp.int32)])
        dst = jnp.concatenate([dst, jnp.full((pad,), n_nodes, jnp.int32)])


    seg_deg = _make_seg_sum(n_acc, n_chunks, with_deg=True)
    seg = _make_seg_sum(n_acc, n_chunks, with_deg=False)

    b1r = b1.reshape(1, D)
    b2r = b2.reshape(1, D)

    y1l, z1r = _mm2(x, W1_l, W1_r)
    acc1, deg_p = seg_deg(src, dst, y1l)
    recip = _deg_recip(deg_p)
    y2l, z2r = _combine_mm2(acc1, recip, b1r, z1r, W2_l, W2_r)
    (acc2,) = seg(src, dst, y2l)
    out = _combine_final(acc2, recip, b2r, z2r)
    return out
